# baseline (speedup 1.0000x reference)
import functools

import jax
import jax.numpy as jnp
from jax import lax
from jax.experimental import pallas as pl
from jax.experimental.pallas import tpu as pltpu
from jax.experimental.pallas import tpu_sc as plsc

D = 768
B = 4
T = 4096

_info = plsc.get_sparse_core_info()
NC, NS, L = _info.num_cores, _info.num_subcores, _info.num_lanes
NW = NC * NS
PW_T = T // NW
CH = 32
HH = CH // 2
NCHUNK = PW_T // CH
NSTEP = NCHUNK * B


def _emb_body(tok_hbm, x_hbm, pos_hbm, out_hbm, idx_v, gbuf, obuf, pos,
              gsem, shsem, psem, isem):
    wid = lax.axis_index("s") * NC + lax.axis_index("c")
    t0 = wid * PW_T

    icopy = [
        pltpu.async_copy(x_hbm.at[b, pl.ds(t0, PW_T)], idx_v.at[b], isem)
        for b in range(B)
    ]

    def start_gather(s, k):
        c, b = s // B, s % B
        return pltpu.async_copy(
            tok_hbm.at[idx_v.at[b, pl.ds(c * CH, CH)]], gbuf[k], gsem[k])

    pcopy = [None] * 2
    pcopy[0] = pltpu.async_copy(pos_hbm.at[pl.ds(t0, CH)], pos[0], psem[0])
    for cp in icopy:
        cp.wait()
    gcopy = [start_gather(0, 0), start_gather(1, 1)]
    scopy = [None, None]

    for s in range(NSTEP):
        k = s % 2
        c, b = s // B, s % B
        q = c % 2
        gcopy[k].wait()
        if b == 0:
            pcopy[q].wait()
            if c + 1 < NCHUNK:
                pcopy[1 - q] = pltpu.async_copy(
                    pos_hbm.at[pl.ds(t0 + (c + 1) * CH, CH)], pos[1 - q], psem[1 - q])

        for h in range(2):
            if scopy[h] is not None:
                scopy[h].wait()

            def row_body(r, carry, _k=k, _q=q, _h=h):
                gr = _h * HH + r
                for j in range(D // L):
                    sl = pl.ds(j * L, L)
                    obuf[_h][r, sl] = gbuf[_k][gr, sl] + pos[_q][gr, sl]
                return carry

            lax.fori_loop(0, HH, row_body, 0)
            scopy[h] = pltpu.async_copy(
                obuf[h], out_hbm.at[b, pl.ds(t0 + c * CH + h * HH, HH)], shsem[h])

        g = s + 2
        if g < NSTEP:
            gcopy[k] = start_gather(g, k)

    for h in range(2):
        scopy[h].wait()


@functools.partial(
    pl.kernel,
    mesh=plsc.VectorSubcoreMesh(core_axis_name="c", subcore_axis_name="s"),
    out_type=jax.ShapeDtypeStruct((B, T, D), jnp.float32),
    scratch_types=[
        pltpu.VMEM((B, PW_T), jnp.int32),
        [pltpu.VMEM((CH, D), jnp.float32) for _ in range(2)],
        [pltpu.VMEM((HH, D), jnp.float32) for _ in range(2)],
        [pltpu.VMEM((CH, D), jnp.float32) for _ in range(2)],
        [pltpu.SemaphoreType.DMA for _ in range(2)],
        [pltpu.SemaphoreType.DMA for _ in range(2)],
        [pltpu.SemaphoreType.DMA for _ in range(2)],
        pltpu.SemaphoreType.DMA,
    ],
)
def _emb_kernel(tok_hbm, x_hbm, pos_hbm, out_hbm, idx_v, gbuf, obuf, pos,
                gsem, shsem, psem, isem):
    _emb_body(tok_hbm, x_hbm, pos_hbm, out_hbm, idx_v, gbuf, obuf, pos,
              gsem, shsem, psem, isem)


def kernel(x, tok_table, pos_table):
    return _emb_kernel(tok_table, x.astype(jnp.int32), pos_table)
